# baseline (speedup 1.0000x reference)
import functools

import jax
import jax.numpy as jnp
from jax import lax
from jax.experimental import pallas as pl
from jax.experimental.pallas import tpu as pltpu
from jax.experimental.pallas import tpu_sc as plsc

_NS = 16
_NW = 32
_CH = 512
_GB = 128

_SC_PARAMS = pltpu.CompilerParams(needs_layout_passes=False,
                                  use_tc_tiling_on_sc=False)


def _cdiv(a, b):
    return (a + b - 1) // b


def _in_body(n, blk, x_ref, w_ref, b_ref, o_ref):
    h = jnp.dot(x_ref[...], w_ref[...], preferred_element_type=jnp.float32)
    h = jnp.maximum(h + b_ref[...], 0.0)
    rows = pl.program_id(0) * blk + lax.broadcasted_iota(jnp.int32, h.shape, 0)
    o_ref[...] = jnp.where(rows < n, h, 0.0)


def _input_mlp(n, xp, w, b):
    np_, d = xp.shape
    h_dim = w.shape[1]
    blk = 1024
    return pl.pallas_call(
        functools.partial(_in_body, n, blk),
        grid=(np_ // blk,),
        in_specs=[
            pl.BlockSpec((blk, d), lambda i: (i, 0)),
            pl.BlockSpec((d, h_dim), lambda i: (0, 0)),
            pl.BlockSpec((1, h_dim), lambda i: (0, 0)),
        ],
        out_specs=pl.BlockSpec((blk, h_dim), lambda i: (i, 0)),
        out_shape=jax.ShapeDtypeStruct((np_, h_dim), jnp.float32),
    )(xp, w, b.reshape(1, h_dim))


def _dinv_body(deg_ref, o_ref):
    cnt = jnp.sum(deg_ref[...], axis=1, keepdims=True) * 0.125
    o_ref[...] = lax.rsqrt(cnt + 1.0)


def _dinv(deg_dense, n_pad):
    return pl.pallas_call(
        _dinv_body,
        out_shape=jax.ShapeDtypeStruct((n_pad, 1), jnp.float32),
    )(deg_dense)


def _pre_body(h_ref, w_ref, dinv_ref, o_ref):
    g = jnp.dot(h_ref[...], w_ref[...], preferred_element_type=jnp.float32)
    o_ref[...] = g * dinv_ref[...]


def _pre(h, w, dinv):
    np_, d = h.shape
    blk = 1024
    return pl.pallas_call(
        _pre_body,
        grid=(np_ // blk,),
        in_specs=[
            pl.BlockSpec((blk, d), lambda i: (i, 0)),
            pl.BlockSpec((d, d), lambda i: (0, 0)),
            pl.BlockSpec((blk, 1), lambda i: (i, 0)),
        ],
        out_specs=pl.BlockSpec((blk, d), lambda i: (i, 0)),
        out_shape=jax.ShapeDtypeStruct((np_, d), jnp.float32),
    )(h, w, dinv)


def _post_body(n, s_ref, g2_ref, dinv_ref, b_ref, bng_ref, bnb_ref, h_ref,
               o_ref):
    t = s_ref[...] + g2_ref[...]
    t = t * dinv_ref[...] + b_ref[...]
    rows = lax.broadcasted_iota(jnp.int32, t.shape, 0)
    mask = rows < n
    inv_n = 1.0 / n
    m = jnp.sum(jnp.where(mask, t, 0.0), axis=0, keepdims=True) * inv_n
    v = jnp.sum(jnp.where(mask, (t - m) ** 2, 0.0), axis=0,
                keepdims=True) * inv_n
    y = (t - m) * lax.rsqrt(v + 1e-5) * bng_ref[...] + bnb_ref[...]
    y = jnp.maximum(y, 0.0) + h_ref[...]
    o_ref[...] = jnp.where(mask, y, 0.0)


def _post(n, s_dense, g2, dinv, b, bng, bnb, h):
    np_, d = h.shape
    return pl.pallas_call(
        functools.partial(_post_body, n),
        out_shape=jax.ShapeDtypeStruct((np_, d), jnp.float32),
    )(s_dense, g2, dinv, b.reshape(1, d), bng.reshape(1, d),
      bnb.reshape(1, d), h)


def _head_body(n, h_ref, w1_ref, b1_ref, g_ref, bb_ref, w2_ref, b2_ref,
               o_ref):
    z = jnp.dot(h_ref[...], w1_ref[...], preferred_element_type=jnp.float32)
    z = z + b1_ref[...]
    rows = lax.broadcasted_iota(jnp.int32, z.shape, 0)
    mask = rows < n
    inv_n = 1.0 / n
    m = jnp.sum(jnp.where(mask, z, 0.0), axis=0, keepdims=True) * inv_n
    v = jnp.sum(jnp.where(mask, (z - m) ** 2, 0.0), axis=0,
                keepdims=True) * inv_n
    z = (z - m) * lax.rsqrt(v + 1e-5) * g_ref[...] + bb_ref[...]
    z = jnp.maximum(z, 0.0)
    o_ref[...] = jnp.dot(z, w2_ref[...],
                         preferred_element_type=jnp.float32) + b2_ref[...]


def _head(n, h, w1, b1, g, bb, w2p, b2p):
    np_, d = h.shape
    cpad = w2p.shape[1]
    return pl.pallas_call(
        functools.partial(_head_body, n),
        out_shape=jax.ShapeDtypeStruct((np_, cpad), jnp.float32),
    )(h, w1, b1.reshape(1, d), g.reshape(1, d), bb.reshape(1, d), w2p,
      b2p.reshape(1, cpad))


def _sc_degree(aidx, zacc, n_pad):
    e_pad = aidx.shape[0] // 16
    epw = e_pad // _NW
    nch = epw // _CH
    aw = n_pad * 8 + 8
    mesh = plsc.VectorSubcoreMesh(core_axis_name="c", subcore_axis_name="s")

    @functools.partial(
        pl.kernel,
        out_type=jax.ShapeDtypeStruct((_NW, n_pad * 8), jnp.float32),
        mesh=mesh,
        scratch_types=[
            pltpu.VMEM((_CH * 16,), jnp.int32),
            pltpu.VMEM((aw,), jnp.float32),
        ],
        compiler_params=_SC_PARAMS,
    )
    def k(aidx_hbm, zacc_hbm, deg_hbm, aibuf, acc):
        c = lax.axis_index("c")
        s = lax.axis_index("s")
        w = c * _NS + s
        pltpu.sync_copy(zacc_hbm, acc)
        lanes = lax.broadcasted_iota(jnp.int32, (16,), 0)
        ones8 = jnp.where(lanes < 8, 1.0, 0.0)

        def chunk(j, _):
            base = (w * epw + j * _CH) * 16
            pltpu.sync_copy(aidx_hbm.at[pl.ds(base, _CH * 16)], aibuf)

            def edge8(e8, _2):
                for u in range(8):
                    e = e8 * 8 + u
                    a16 = aibuf[pl.ds(e * 16, 16)]
                    plsc.addupdate_scatter(acc, [a16], ones8)
                return 0

            lax.fori_loop(0, _CH // 8, edge8, 0)
            return 0

        lax.fori_loop(0, nch, chunk, 0)
        pltpu.sync_copy(acc.at[pl.ds(0, n_pad * 8)], deg_hbm.at[w])

    return k(aidx, zacc)


def _sc_scatter(g16, src1, aidx, zacc, n_pad):
    e_pad = src1.shape[0]
    nch = e_pad // _CH
    aw = n_pad * 8 + 8
    mesh = plsc.VectorSubcoreMesh(core_axis_name="c", subcore_axis_name="s")

    @functools.partial(
        pl.kernel,
        out_type=jax.ShapeDtypeStruct((_NW, n_pad * 8), jnp.float32),
        mesh=mesh,
        scratch_types=[
            pltpu.VMEM((_CH,), jnp.int32),
            pltpu.VMEM((_CH,), jnp.int32),
            pltpu.VMEM((_CH * 16,), jnp.int32),
            pltpu.VMEM((_CH * 16,), jnp.int32),
            pltpu.VMEM((_CH, 16), jnp.float32),
            pltpu.VMEM((_CH, 16), jnp.float32),
            pltpu.VMEM((aw,), jnp.float32),
            pltpu.SemaphoreType.DMA,
            pltpu.SemaphoreType.DMA,
        ],
        compiler_params=_SC_PARAMS,
    )
    def k(g16_hbm, src_hbm, aidx_hbm, zacc_hbm, s_hbm, si0, si1, ai0, ai1,
          rb0, rb1, acc, sem0, sem1):
        c = lax.axis_index("c")
        s = lax.axis_index("s")
        w = c * _NS + s
        gt = g16_hbm.at[w]
        pltpu.sync_copy(zacc_hbm, acc)

        def stage(ch, si, ai, sem):
            pltpu.async_copy(src_hbm.at[pl.ds(ch * _CH, _CH)], si, sem)
            pltpu.async_copy(aidx_hbm.at[pl.ds(ch * _CH * 16, _CH * 16)],
                             ai, sem)

        def wait_stage(ch, si, ai, sem):
            pltpu.make_async_copy(src_hbm.at[pl.ds(ch * _CH, _CH)], si,
                                  sem).wait()
            pltpu.make_async_copy(
                aidx_hbm.at[pl.ds(ch * _CH * 16, _CH * 16)], ai, sem).wait()

        def fire(si, rb, sem):
            for kk in range(_CH // _GB):
                pltpu.async_copy(gt.at[si.at[pl.ds(kk * _GB, _GB)]],
                                 rb.at[pl.ds(kk * _GB, _GB)], sem)

        def wait_gather(si, rb, sem):
            for kk in range(_CH // _GB):
                pltpu.make_async_copy(gt.at[si.at[pl.ds(kk * _GB, _GB)]],
                                      rb.at[pl.ds(kk * _GB, _GB)],
                                      sem).wait()

        def compute(ai, rb):
            def edge16(e16, _2):
                for u in range(16):
                    e = e16 * 16 + u
                    a16 = ai[pl.ds(e * 16, 16)]
                    v16 = rb[e, :]
                    plsc.addupdate_scatter(acc, [a16], v16)
                return 0

            lax.fori_loop(0, _CH // 16, edge16, 0)

        stage(0, si0, ai0, sem0)
        wait_stage(0, si0, ai0, sem0)
        fire(si0, rb0, sem0)
        stage(1, si1, ai1, sem1)
        wait_stage(1, si1, ai1, sem1)

        def pair(i, _):
            cha = 2 * i
            chb = 2 * i + 1
            fire(si1, rb1, sem1)
            wait_gather(si0, rb0, sem0)
            compute(ai0, rb0)
            stage(jnp.minimum(cha + 2, nch - 1), si0, ai0, sem0)
            wait_gather(si1, rb1, sem1)
            compute(ai1, rb1)
            stage(jnp.minimum(chb + 2, nch - 1), si1, ai1, sem1)
            wait_stage(cha, si0, ai0, sem0)
            fire(si0, rb0, sem0)
            wait_stage(chb, si1, ai1, sem1)
            return 0

        lax.fori_loop(0, nch // 2, pair, 0)
        wait_gather(si0, rb0, sem0)

        pltpu.sync_copy(acc.at[pl.ds(0, n_pad * 8)], s_hbm.at[w])

    return k(g16, src1, aidx, zacc)


def kernel(x, edge_index, W_in, b_in, conv_W, conv_b, bn_g, bn_b,
           W_h1, b_h1, h_g, h_b, W_h2, b_h2):
    n = x.shape[0]
    e = edge_index.shape[1]
    num_layers = conv_W.shape[0]
    d = W_in.shape[1]
    c_out = W_h2.shape[1]

    n_pad = _cdiv(n + 1, 1024) * 1024
    e_pad = _cdiv(e, _NW * _CH) * _NW * _CH
    npad_e = e_pad - e

    src = edge_index[0]
    dst = edge_index[1]
    pi = jnp.arange(npad_e, dtype=jnp.int32)
    src_p = jnp.concatenate([src, pi % n])
    dst_p = jnp.concatenate([dst, n + pi % (n_pad - n)])

    lane = jnp.arange(16, dtype=jnp.int32)
    addr = jnp.where(lane[None, :] < 8,
                     dst_p[:, None] * 8 + lane[None, :],
                     n_pad * 8 + (lane[None, :] - 8))
    aidx = addr.reshape(-1)
    zacc = jnp.zeros((n_pad * 8 + 8,), jnp.float32)

    xp = jnp.pad(x, ((0, n_pad - n), (0, 0)))

    def to_stripes(dense):
        st = dense.reshape(n_pad, _NW, 8).transpose(1, 0, 2)
        return jnp.pad(st, ((0, 0), (0, 0), (0, 8)))

    def from_stripes(flat):
        return flat.reshape(_NW, n_pad, 8).transpose(1, 0, 2).reshape(
            n_pad, _NW * 8)

    deg_dense = from_stripes(_sc_degree(aidx, zacc, n_pad))
    dinv = _dinv(deg_dense, n_pad)

    h = _input_mlp(n, xp, W_in, b_in)
    for i in range(num_layers):
        g2 = _pre(h, conv_W[i], dinv)
        g16 = to_stripes(g2)
        s_dense = from_stripes(_sc_scatter(g16, src_p, aidx, zacc, n_pad))
        h = _post(n, s_dense, g2, dinv, conv_b[i], bn_g[i], bn_b[i], h)

    cpad = _cdiv(c_out, 128) * 128
    w2p = jnp.pad(W_h2, ((0, 0), (0, cpad - c_out)))
    b2p = jnp.pad(b_h2, (0, cpad - c_out))
    out = _head(n, h, W_h1, b_h1, h_g, h_b, w2p, b2p)
    return out[:n, :c_out]


# speedup vs baseline: 1.5577x; 1.5577x over previous
import functools

import jax
import jax.numpy as jnp
from jax import lax
from jax.experimental import pallas as pl
from jax.experimental.pallas import tpu as pltpu
from jax.experimental.pallas import tpu_sc as plsc

_NS = 16
_NW = 32
_CH = 512
_GB = 128

_SC_PARAMS = pltpu.CompilerParams(needs_layout_passes=False,
                                  use_tc_tiling_on_sc=False)


def _cdiv(a, b):
    return (a + b - 1) // b


def _in_body(n, blk, x_ref, w_ref, b_ref, o_ref):
    h = jnp.dot(x_ref[...], w_ref[...], preferred_element_type=jnp.float32)
    h = jnp.maximum(h + b_ref[...], 0.0)
    rows = pl.program_id(0) * blk + lax.broadcasted_iota(jnp.int32, h.shape, 0)
    o_ref[...] = jnp.where(rows < n, h, 0.0)


def _input_mlp(n, xp, w, b):
    np_, d = xp.shape
    h_dim = w.shape[1]
    blk = 1024
    return pl.pallas_call(
        functools.partial(_in_body, n, blk),
        grid=(np_ // blk,),
        in_specs=[
            pl.BlockSpec((blk, d), lambda i: (i, 0)),
            pl.BlockSpec((d, h_dim), lambda i: (0, 0)),
            pl.BlockSpec((1, h_dim), lambda i: (0, 0)),
        ],
        out_specs=pl.BlockSpec((blk, h_dim), lambda i: (i, 0)),
        out_shape=jax.ShapeDtypeStruct((np_, h_dim), jnp.float32),
    )(xp, w, b.reshape(1, h_dim))


def _dinv_body(deg_ref, o_ref):
    cnt = jnp.sum(deg_ref[...], axis=1, keepdims=True) * 0.125
    o_ref[...] = lax.rsqrt(cnt + 1.0)


def _dinv(deg_dense, n_pad):
    return pl.pallas_call(
        _dinv_body,
        out_shape=jax.ShapeDtypeStruct((n_pad, 1), jnp.float32),
    )(deg_dense)


def _pre_body(h_ref, w_ref, dinv_ref, o_ref):
    g = jnp.dot(h_ref[...], w_ref[...], preferred_element_type=jnp.float32)
    o_ref[...] = g * dinv_ref[...]


def _pre(h, w, dinv):
    np_, d = h.shape
    blk = 1024
    return pl.pallas_call(
        _pre_body,
        grid=(np_ // blk,),
        in_specs=[
            pl.BlockSpec((blk, d), lambda i: (i, 0)),
            pl.BlockSpec((d, d), lambda i: (0, 0)),
            pl.BlockSpec((blk, 1), lambda i: (i, 0)),
        ],
        out_specs=pl.BlockSpec((blk, d), lambda i: (i, 0)),
        out_shape=jax.ShapeDtypeStruct((np_, d), jnp.float32),
    )(h, w, dinv)


def _post_body(n, s_ref, g2_ref, dinv_ref, b_ref, bng_ref, bnb_ref, h_ref,
               o_ref):
    t = s_ref[...] + g2_ref[...]
    t = t * dinv_ref[...] + b_ref[...]
    rows = lax.broadcasted_iota(jnp.int32, t.shape, 0)
    mask = rows < n
    inv_n = 1.0 / n
    m = jnp.sum(jnp.where(mask, t, 0.0), axis=0, keepdims=True) * inv_n
    v = jnp.sum(jnp.where(mask, (t - m) ** 2, 0.0), axis=0,
                keepdims=True) * inv_n
    y = (t - m) * lax.rsqrt(v + 1e-5) * bng_ref[...] + bnb_ref[...]
    y = jnp.maximum(y, 0.0) + h_ref[...]
    o_ref[...] = jnp.where(mask, y, 0.0)


def _post(n, s_dense, g2, dinv, b, bng, bnb, h):
    np_, d = h.shape
    return pl.pallas_call(
        functools.partial(_post_body, n),
        out_shape=jax.ShapeDtypeStruct((np_, d), jnp.float32),
    )(s_dense, g2, dinv, b.reshape(1, d), bng.reshape(1, d),
      bnb.reshape(1, d), h)


def _head_body(n, h_ref, w1_ref, b1_ref, g_ref, bb_ref, w2_ref, b2_ref,
               o_ref):
    z = jnp.dot(h_ref[...], w1_ref[...], preferred_element_type=jnp.float32)
    z = z + b1_ref[...]
    rows = lax.broadcasted_iota(jnp.int32, z.shape, 0)
    mask = rows < n
    inv_n = 1.0 / n
    m = jnp.sum(jnp.where(mask, z, 0.0), axis=0, keepdims=True) * inv_n
    v = jnp.sum(jnp.where(mask, (z - m) ** 2, 0.0), axis=0,
                keepdims=True) * inv_n
    z = (z - m) * lax.rsqrt(v + 1e-5) * g_ref[...] + bb_ref[...]
    z = jnp.maximum(z, 0.0)
    o_ref[...] = jnp.dot(z, w2_ref[...],
                         preferred_element_type=jnp.float32) + b2_ref[...]


def _head(n, h, w1, b1, g, bb, w2p, b2p):
    np_, d = h.shape
    cpad = w2p.shape[1]
    return pl.pallas_call(
        functools.partial(_head_body, n),
        out_shape=jax.ShapeDtypeStruct((np_, cpad), jnp.float32),
    )(h, w1, b1.reshape(1, d), g.reshape(1, d), bb.reshape(1, d), w2p,
      b2p.reshape(1, cpad))


def _sc_degree(aidx, zacc, n_pad):
    e_pad = aidx.shape[0] // 16
    epw = e_pad // _NW
    nch = epw // _CH
    aw = n_pad * 8 + 8
    mesh = plsc.VectorSubcoreMesh(core_axis_name="c", subcore_axis_name="s")

    @functools.partial(
        pl.kernel,
        out_type=jax.ShapeDtypeStruct((_NW, n_pad * 8), jnp.float32),
        mesh=mesh,
        scratch_types=[
            pltpu.VMEM((_CH * 16,), jnp.int32),
            pltpu.VMEM((aw,), jnp.float32),
        ],
        compiler_params=_SC_PARAMS,
    )
    def k(aidx_hbm, zacc_hbm, deg_hbm, aibuf, acc):
        c = lax.axis_index("c")
        s = lax.axis_index("s")
        w = c * _NS + s
        pltpu.sync_copy(zacc_hbm, acc)
        lanes = lax.broadcasted_iota(jnp.int32, (16,), 0)
        ones8 = jnp.where(lanes < 8, 1.0, 0.0)

        def chunk(j, _):
            base = (w * epw + j * _CH) * 16
            pltpu.sync_copy(aidx_hbm.at[pl.ds(base, _CH * 16)], aibuf)

            def edge8(e8, _2):
                for u in range(8):
                    e = e8 * 8 + u
                    a16 = aibuf[pl.ds(e * 16, 16)]
                    plsc.addupdate_scatter(acc, [a16], ones8)
                return 0

            lax.fori_loop(0, _CH // 8, edge8, 0)
            return 0

        lax.fori_loop(0, nch, chunk, 0)
        pltpu.sync_copy(acc.at[pl.ds(0, n_pad * 8)], deg_hbm.at[w])

    return k(aidx, zacc)


def _sc_scatter(g16, src1, aidx, zacc, n_pad):
    e_pad = src1.shape[0]
    nch = e_pad // _CH
    aw = n_pad * 8 + 8
    mesh = plsc.VectorSubcoreMesh(core_axis_name="c", subcore_axis_name="s")

    @functools.partial(
        pl.kernel,
        out_type=jax.ShapeDtypeStruct((_NW, n_pad * 8), jnp.float32),
        mesh=mesh,
        scratch_types=[
            pltpu.VMEM((_CH,), jnp.int32),
            pltpu.VMEM((_CH,), jnp.int32),
            pltpu.VMEM((_CH * 16,), jnp.int32),
            pltpu.VMEM((_CH * 16,), jnp.int32),
            pltpu.VMEM((_CH, 16), jnp.float32),
            pltpu.VMEM((_CH, 16), jnp.float32),
            pltpu.VMEM((aw,), jnp.float32),
            pltpu.SemaphoreType.DMA,
            pltpu.SemaphoreType.DMA,
        ],
        compiler_params=_SC_PARAMS,
    )
    def k(g16_hbm, src_hbm, aidx_hbm, zacc_hbm, s_hbm, si0, si1, ai0, ai1,
          rb0, rb1, acc, sem0, sem1):
        c = lax.axis_index("c")
        s = lax.axis_index("s")
        w = c * _NS + s
        gt = g16_hbm.at[w]
        pltpu.sync_copy(zacc_hbm, acc)

        def stage(ch, si, ai, sem):
            pltpu.async_copy(src_hbm.at[pl.ds(ch * _CH, _CH)], si, sem)
            pltpu.async_copy(aidx_hbm.at[pl.ds(ch * _CH * 16, _CH * 16)],
                             ai, sem)

        def wait_stage(ch, si, ai, sem):
            pltpu.make_async_copy(src_hbm.at[pl.ds(ch * _CH, _CH)], si,
                                  sem).wait()
            pltpu.make_async_copy(
                aidx_hbm.at[pl.ds(ch * _CH * 16, _CH * 16)], ai, sem).wait()

        def fire(si, rb, sem):
            for kk in range(_CH // _GB):
                pltpu.async_copy(gt.at[si.at[pl.ds(kk * _GB, _GB)]],
                                 rb.at[pl.ds(kk * _GB, _GB)], sem)

        def wait_gather(si, rb, sem):
            for kk in range(_CH // _GB):
                pltpu.make_async_copy(gt.at[si.at[pl.ds(kk * _GB, _GB)]],
                                      rb.at[pl.ds(kk * _GB, _GB)],
                                      sem).wait()

        def compute(ai, rb):
            @plsc.parallel_loop(0, _CH, 1, unroll=8)
            def _(e):
                a16 = ai[pl.ds(e * 16, 16)]
                v16 = rb[e, :]
                plsc.addupdate_scatter(acc, [a16], v16)

        stage(0, si0, ai0, sem0)
        wait_stage(0, si0, ai0, sem0)
        fire(si0, rb0, sem0)
        stage(1, si1, ai1, sem1)
        wait_stage(1, si1, ai1, sem1)

        def pair(i, _):
            cha = 2 * i
            chb = 2 * i + 1
            fire(si1, rb1, sem1)
            wait_gather(si0, rb0, sem0)
            compute(ai0, rb0)
            stage(jnp.minimum(cha + 2, nch - 1), si0, ai0, sem0)
            wait_gather(si1, rb1, sem1)
            compute(ai1, rb1)
            stage(jnp.minimum(chb + 2, nch - 1), si1, ai1, sem1)
            wait_stage(cha, si0, ai0, sem0)
            fire(si0, rb0, sem0)
            wait_stage(chb, si1, ai1, sem1)
            return 0

        lax.fori_loop(0, nch // 2, pair, 0)
        wait_gather(si0, rb0, sem0)

        pltpu.sync_copy(acc.at[pl.ds(0, n_pad * 8)], s_hbm.at[w])

    return k(g16, src1, aidx, zacc)


def kernel(x, edge_index, W_in, b_in, conv_W, conv_b, bn_g, bn_b,
           W_h1, b_h1, h_g, h_b, W_h2, b_h2):
    n = x.shape[0]
    e = edge_index.shape[1]
    num_layers = conv_W.shape[0]
    d = W_in.shape[1]
    c_out = W_h2.shape[1]

    n_pad = _cdiv(n + 1, 1024) * 1024
    e_pad = _cdiv(e, _NW * _CH) * _NW * _CH
    npad_e = e_pad - e

    src = edge_index[0]
    dst = edge_index[1]
    pi = jnp.arange(npad_e, dtype=jnp.int32)
    src_p = jnp.concatenate([src, pi % n])
    dst_p = jnp.concatenate([dst, n + pi % (n_pad - n)])

    lane = jnp.arange(16, dtype=jnp.int32)
    addr = jnp.where(lane[None, :] < 8,
                     dst_p[:, None] * 8 + lane[None, :],
                     n_pad * 8 + (lane[None, :] - 8))
    aidx = addr.reshape(-1)
    zacc = jnp.zeros((n_pad * 8 + 8,), jnp.float32)

    xp = jnp.pad(x, ((0, n_pad - n), (0, 0)))

    def to_stripes(dense):
        st = dense.reshape(n_pad, _NW, 8).transpose(1, 0, 2)
        return jnp.pad(st, ((0, 0), (0, 0), (0, 8)))

    def from_stripes(flat):
        return flat.reshape(_NW, n_pad, 8).transpose(1, 0, 2).reshape(
            n_pad, _NW * 8)

    deg_dense = from_stripes(_sc_degree(aidx, zacc, n_pad))
    dinv = _dinv(deg_dense, n_pad)

    h = _input_mlp(n, xp, W_in, b_in)
    for i in range(num_layers):
        g2 = _pre(h, conv_W[i], dinv)
        g16 = to_stripes(g2)
        s_dense = from_stripes(_sc_scatter(g16, src_p, aidx, zacc, n_pad))
        h = _post(n, s_dense, g2, dinv, conv_b[i], bn_g[i], bn_b[i], h)

    cpad = _cdiv(c_out, 128) * 128
    w2p = jnp.pad(W_h2, ((0, 0), (0, cpad - c_out)))
    b2p = jnp.pad(b_h2, (0, cpad - c_out))
    out = _head(n, h, W_h1, b_h1, h_g, h_b, w2p, b2p)
    return out[:n, :c_out]
